# baseline (speedup 1.0000x reference)
import functools

import jax
import jax.numpy as jnp
from jax import lax
from jax.experimental import pallas as pl
from jax.experimental.pallas import tpu as pltpu
from jax.experimental.pallas import tpu_sc as plsc

_B = 16384
_EMB = 64
_NW = 32
_BPW = _B // _NW
_NCH = 4
_CH = 128

_f32 = jnp.float32


def _sc_main(uidx, bidx, uemb2, bemb2, ubt1, bbt1):
  mesh = plsc.VectorSubcoreMesh(
      core_axis_name="c", subcore_axis_name="s", num_cores=2, num_subcores=16)

  @functools.partial(
      pl.kernel,
      out_type=(
          jax.ShapeDtypeStruct((_NW, 128), _f32),
          jax.ShapeDtypeStruct((128, 128), _f32),
          jax.ShapeDtypeStruct((128, 128), _f32),
      ),
      mesh=mesh,
      compiler_params=pltpu.CompilerParams(
          use_tc_tiling_on_sc=True, needs_layout_passes=False),
      scratch_types=[
          pltpu.VMEM((_NCH, _CH), jnp.int32),
          pltpu.VMEM((_NCH, _CH), jnp.int32),
          pltpu.VMEM((_NCH, _CH), jnp.int32),
          pltpu.VMEM((_NCH, _CH), jnp.int32),
          pltpu.VMEM((_NCH, _CH), jnp.int32),
          pltpu.VMEM((_NCH, _CH), jnp.int32),
          pltpu.VMEM((_CH, 128), _f32),
          pltpu.VMEM((_CH, 128), _f32),
          pltpu.VMEM((_CH, 128), _f32),
          pltpu.VMEM((_CH, 128), _f32),
          pltpu.VMEM((_NCH, _CH), _f32),
          pltpu.VMEM((_NCH, _CH), _f32),
          pltpu.VMEM((128,), _f32),
          pltpu.SemaphoreType.DMA,
          pltpu.SemaphoreType.DMA,
      ],
  )
  def sc_k(uidx_h, bidx_h, uemb_h, bemb_h, ubt_h, bbt_h,
           part_o, ub_o, bb_o,
           uidx_v, bidx_v, urow_v, brow_v, ucol_v, bcol_v,
           ubuf0, ubuf1, bbuf0, bbuf1, ubias_v, bbias_v, accv, sem_e, sem_b):
    ubufs = (ubuf0, ubuf1)
    bbufs = (bbuf0, bbuf1)
    wid = lax.axis_index("s") * 2 + lax.axis_index("c")
    row0 = wid * _NCH
    pltpu.sync_copy(uidx_h.at[pl.ds(row0, _NCH)], uidx_v)
    pltpu.sync_copy(bidx_h.at[pl.ds(row0, _NCH)], bidx_v)

    for j in range(_NCH):
      for t in range(_CH // 16):
        sl = pl.ds(t * 16, 16)
        iu = uidx_v[j, sl]
        ib = bidx_v[j, sl]
        urow_v[j, sl] = lax.shift_right_logical(iu, 1)
        brow_v[j, sl] = lax.shift_right_logical(ib, 1)
        ucol_v[j, sl] = lax.shift_left(jnp.bitwise_and(iu, 1), 6)
        bcol_v[j, sl] = lax.shift_left(jnp.bitwise_and(ib, 1), 6)

    bias_copies = []
    for j in range(_NCH):
      bias_copies.append(
          pltpu.async_copy(ubt_h.at[uidx_v.at[j]], ubias_v.at[j], sem_b))
      bias_copies.append(
          pltpu.async_copy(bbt_h.at[bidx_v.at[j]], bbias_v.at[j], sem_b))

    def fire(j):
      buf = j % 2
      return (
          pltpu.async_copy(uemb_h.at[urow_v.at[j]], ubufs[buf], sem_e),
          pltpu.async_copy(bemb_h.at[brow_v.at[j]], bbufs[buf], sem_e),
      )

    pending = fire(0)
    lanes = lax.iota(jnp.int32, 16)

    acc = jnp.zeros((16,), _f32)
    for j in range(_NCH):
      for c in pending:
        c.wait()
      if j + 1 < _NCH:
        pending = fire(j + 1)
      ub_ref, bb_ref = ubufs[j % 2], bbufs[j % 2]
      for g in range(_CH // 16):
        rows = lanes + (g * 16)
        ucol = ucol_v[j, pl.ds(g * 16, 16)]
        bcol = bcol_v[j, pl.ds(g * 16, 16)]

        def mbody(m, a, ub_ref=ub_ref, bb_ref=bb_ref,
                  rows=rows, ucol=ucol, bcol=bcol):
          m2 = m * 2
          uu0 = plsc.load_gather(ub_ref, [rows, ucol + m2])
          vv0 = plsc.load_gather(bb_ref, [rows, bcol + m2])
          uu1 = plsc.load_gather(ub_ref, [rows, ucol + (m2 + 1)])
          vv1 = plsc.load_gather(bb_ref, [rows, bcol + (m2 + 1)])
          return a + uu0 * vv0 + uu1 * vv1

        acc = lax.fori_loop(0, _EMB // 2, mbody, acc)

    accv[pl.ds(0, 16)] = acc
    for t in range(1, 8):
      accv[pl.ds(t * 16, 16)] = jnp.zeros((16,), _f32)
    for c in bias_copies:
      c.wait()

    pltpu.sync_copy(accv, part_o.at[wid])
    out_sl = pl.ds(row0, _NCH)
    pltpu.sync_copy(ubias_v, ub_o.at[out_sl])
    pltpu.sync_copy(bbias_v, bb_o.at[out_sl])

  return sc_k(uidx, bidx, uemb2, bemb2, ubt1, bbt1)


def _tc_body(part_ref, ub_ref, bb_ref, o_ref):
  s = jnp.sum(part_ref[...])
  o_ref[...] = jax.nn.sigmoid(ub_ref[...] + bb_ref[...] + s)


def kernel(inputs, user_embedding, user_bias_table, book_embedding,
           book_bias_table):
  idx = inputs.astype(jnp.int32)
  uidx = idx[:, 0].reshape(128, 128)
  bidx = idx[:, 1].reshape(128, 128)
  uemb2 = user_embedding.reshape(50000, 128)
  bemb2 = book_embedding.reshape(50000, 128)
  ubt1 = user_bias_table.reshape(100000)
  bbt1 = book_bias_table.reshape(100000)
  partials, ub, bb = _sc_main(uidx, bidx, uemb2, bemb2, ubt1, bbt1)
  out = pl.pallas_call(
      _tc_body,
      out_shape=jax.ShapeDtypeStruct((128, 128), _f32),
  )(partials, ub, bb)
  return out.reshape(_B, 1)


# speedup vs baseline: 1.5593x; 1.5593x over previous
import functools

import jax
import jax.numpy as jnp
from jax import lax
from jax.experimental import pallas as pl
from jax.experimental.pallas import tpu as pltpu
from jax.experimental.pallas import tpu_sc as plsc

_B = 16384
_EMB = 64
_NW = 32
_BPW = _B // _NW
_NCH = 4
_CH = 128

_f32 = jnp.float32


def _sc_main(uidx, bidx, uemb, bemb, ubt1, bbt1):
  mesh = plsc.VectorSubcoreMesh(
      core_axis_name="c", subcore_axis_name="s", num_cores=2, num_subcores=16)

  @functools.partial(
      pl.kernel,
      out_type=(
          jax.ShapeDtypeStruct((_NW, 128), _f32),
          jax.ShapeDtypeStruct((128, 128), _f32),
          jax.ShapeDtypeStruct((128, 128), _f32),
      ),
      mesh=mesh,
      compiler_params=pltpu.CompilerParams(
          use_tc_tiling_on_sc=True, needs_layout_passes=False),
      scratch_types=[
          pltpu.VMEM((_NCH, _CH), jnp.int32),
          pltpu.VMEM((_NCH, _CH), jnp.int32),
          pltpu.VMEM((_CH, _EMB), _f32),
          pltpu.VMEM((_CH, _EMB), _f32),
          pltpu.VMEM((_CH, _EMB), _f32),
          pltpu.VMEM((_CH, _EMB), _f32),
          pltpu.VMEM((_NCH, _CH), _f32),
          pltpu.VMEM((_NCH, _CH), _f32),
          pltpu.VMEM((128,), _f32),
          pltpu.SemaphoreType.DMA,
          pltpu.SemaphoreType.DMA,
          pltpu.SemaphoreType.DMA,
          pltpu.SemaphoreType.DMA,
          pltpu.SemaphoreType.DMA,
      ],
  )
  def sc_k(uidx_h, bidx_h, uemb_h, bemb_h, ubt_h, bbt_h,
           part_o, ub_o, bb_o,
           uidx_v, bidx_v,
           u0, u1, b0, b1, ubias_v, bbias_v, accv,
           sem_u0, sem_u1, sem_b0, sem_b1, sem_bias):
    ubufs, bbufs = (u0, u1), (b0, b1)
    usems, bsems = (sem_u0, sem_u1), (sem_b0, sem_b1)
    wid = lax.axis_index("s") * 2 + lax.axis_index("c")
    row0 = wid * _NCH
    pltpu.sync_copy(uidx_h.at[pl.ds(row0, _NCH)], uidx_v)
    pltpu.sync_copy(bidx_h.at[pl.ds(row0, _NCH)], bidx_v)

    bias_copies = []
    for j in range(_NCH):
      bias_copies.append(
          pltpu.async_copy(ubt_h.at[uidx_v.at[j]], ubias_v.at[j], sem_bias))
      bias_copies.append(
          pltpu.async_copy(bbt_h.at[bidx_v.at[j]], bbias_v.at[j], sem_bias))

    def issue_chunk(j):
      bu, bb2 = ubufs[j % 2], bbufs[j % 2]
      su, sb = usems[j % 2], bsems[j % 2]

      def it(g, _):
        base = g * 16
        u16 = uidx_v[j, pl.ds(base, 16)]
        b16 = bidx_v[j, pl.ds(base, 16)]
        for t in range(16):
          pltpu.async_copy(uemb_h.at[u16[t]], bu.at[base + t], su)
          pltpu.async_copy(bemb_h.at[b16[t]], bb2.at[base + t], sb)
        return 0

      lax.fori_loop(0, _CH // 16, it, 0)

    def drain_chunk(j):
      pltpu.make_async_copy(
          uemb_h.at[pl.ds(0, _CH)], ubufs[j % 2], usems[j % 2]).wait()
      pltpu.make_async_copy(
          bemb_h.at[pl.ds(0, _CH)], bbufs[j % 2], bsems[j % 2]).wait()

    issue_chunk(0)
    acc = jnp.zeros((16,), _f32)
    for j in range(_NCH):
      drain_chunk(j)
      if j + 1 < _NCH:
        issue_chunk(j + 1)
      bu, bb2 = ubufs[j % 2], bbufs[j % 2]

      def body(r, a, bu=bu, bb2=bb2):
        for k in range(_EMB // 16):
          sl = pl.ds(k * 16, 16)
          a = a + bu[r, sl] * bb2[r, sl]
        return a

      acc = lax.fori_loop(0, _CH, body, acc)

    accv[pl.ds(0, 16)] = acc
    for t in range(1, 8):
      accv[pl.ds(t * 16, 16)] = jnp.zeros((16,), _f32)
    for c in bias_copies:
      c.wait()

    pltpu.sync_copy(accv, part_o.at[wid])
    out_sl = pl.ds(row0, _NCH)
    pltpu.sync_copy(ubias_v, ub_o.at[out_sl])
    pltpu.sync_copy(bbias_v, bb_o.at[out_sl])

  return sc_k(uidx, bidx, uemb, bemb, ubt1, bbt1)


def _tc_body(part_ref, ub_ref, bb_ref, o_ref):
  s = jnp.sum(part_ref[...])
  o_ref[...] = jax.nn.sigmoid(ub_ref[...] + bb_ref[...] + s)


def kernel(inputs, user_embedding, user_bias_table, book_embedding,
           book_bias_table):
  idx = inputs.astype(jnp.int32)
  uidx = idx[:, 0].reshape(128, 128)
  bidx = idx[:, 1].reshape(128, 128)
  ubt1 = user_bias_table.reshape(100000)
  bbt1 = book_bias_table.reshape(100000)
  partials, ub, bb = _sc_main(uidx, bidx, user_embedding, book_embedding,
                              ubt1, bbt1)
  out = pl.pallas_call(
      _tc_body,
      out_shape=jax.ShapeDtypeStruct((128, 128), _f32),
  )(partials, ub, bb)
  return out.reshape(_B, 1)
